# Initial kernel scaffold; baseline (speedup 1.0000x reference)
import jax
import jax.numpy as jnp
from jax.experimental import pallas as pl


def kernel(x, edge_index, W1, a_src1, a_dst1, b1, W2, a_src2, a_dst2, b2, W3, a_src3, a_dst3, b3, Wr, br):
    raise NotImplementedError("write your pallas kernel here")


# speedup vs baseline: 10.0935x; 10.0935x over previous
import functools

import jax
import jax.numpy as jnp
from jax import lax
from jax.experimental import pallas as pl
from jax.experimental.pallas import tpu as pltpu
from jax.experimental.pallas import tpu_sc as plsc

N = 10000
E = 320000
D_IN = 128
HID = 128
HEADS = 8

NW = 32
NB = 320
NPAD = NW * NB
BN = 512
NBLK = NPAD // BN
KMAXS = 11264
CB2 = 256
EPAD = E + KMAXS + 32

def _mesh():
    return plsc.VectorSubcoreMesh(core_axis_name="c", subcore_axis_name="s")


def _dense_body(apply_elu, x_ref, b_ref, W_ref, a_s_ref, a_d_ref,
                h_ref, as_ref, ad_ref):
    xb = x_ref[...]
    if apply_elu:
        xb = xb + b_ref[0]
        xb = jnp.where(xb > 0, xb, jnp.exp(jnp.minimum(xb, 0.0)) - 1.0)
    h = jnp.dot(xb, W_ref[...], preferred_element_type=jnp.float32)
    h_ref[0] = h
    as_ref[0, 0] = jnp.sum(h * a_s_ref[0], axis=1)
    ad_ref[0, 0] = jnp.sum(h * a_d_ref[0], axis=1)


def _dense_layer(x, b_prev, W, a_s, a_d, heads, dout, apply_elu):
    din = x.shape[1]
    grid = (NBLK, heads)
    out_shapes = [
        jax.ShapeDtypeStruct((heads, NPAD, dout), jnp.float32),
        jax.ShapeDtypeStruct((heads, 1, NPAD), jnp.float32),
        jax.ShapeDtypeStruct((heads, 1, NPAD), jnp.float32),
    ]
    fn = pl.pallas_call(
        functools.partial(_dense_body, apply_elu),
        grid=grid,
        in_specs=[
            pl.BlockSpec((BN, din), lambda i, h: (i, 0)),
            pl.BlockSpec((1, din), lambda i, h: (0, 0)),
            pl.BlockSpec((din, dout), lambda i, h: (0, h)),
            pl.BlockSpec((1, 1, dout), lambda i, h: (h, 0, 0)),
            pl.BlockSpec((1, 1, dout), lambda i, h: (h, 0, 0)),
        ],
        out_specs=[
            pl.BlockSpec((1, BN, dout), lambda i, h: (h, i, 0)),
            pl.BlockSpec((1, 1, BN), lambda i, h: (h, 0, i)),
            pl.BlockSpec((1, 1, BN), lambda i, h: (h, 0, i)),
        ],
        out_shape=out_shapes,
    )
    return fn(x, b_prev.reshape(1, din), W, a_s.reshape(heads, 1, dout),
              a_d.reshape(heads, 1, dout))


def _final_body(x_ref, b_ref, w_ref, br_ref, y_ref):
    xb = x_ref[...] + b_ref[0]
    xb = jnp.where(xb > 0, xb, jnp.exp(jnp.minimum(xb, 0.0)) - 1.0)
    y = jnp.sum(xb * w_ref[0], axis=1) + br_ref[0, 0]
    y_ref[0, 0] = jax.nn.sigmoid(y)


def _final_layer(x, b_prev, Wr, br):
    din = x.shape[1]
    fn = pl.pallas_call(
        _final_body,
        grid=(NBLK,),
        in_specs=[
            pl.BlockSpec((BN, din), lambda i: (i, 0)),
            pl.BlockSpec((1, din), lambda i: (0, 0)),
            pl.BlockSpec((1, din), lambda i: (0, 0)),
            pl.BlockSpec((1, 1), lambda i: (0, 0)),
        ],
        out_specs=pl.BlockSpec((1, 1, BN), lambda i: (i, 0, 0)),
        out_shape=jax.ShapeDtypeStruct((NBLK, 1, BN), jnp.float32),
    )
    y = fn(x, b_prev.reshape(1, din), Wr.reshape(1, din),
           br.reshape(1, 1))
    return y.reshape(NPAD)


def _edge_body(H, D, DR, h_t, asrc_t, adst_t, srcs, dsts, bnds, agg,
               bv, srcb, dstb, asrc_v, adst_v, den_v, attnb, idxb,
               rows_v, acc, sem):
    wid = lax.axis_index("s") * 2 + lax.axis_index("c")
    n0 = wid * NB
    pltpu.sync_copy(bnds, bv)
    ii = lax.iota(jnp.int32, 16)

    def _bnd(w):
        chunk = bv[pl.ds((w // 16) * 16, 16)]
        return jnp.sum(jnp.where(ii == w % 16, chunk, 0))

    e0 = _bnd(wid)
    e1 = _bnd(wid + 1)
    ea = (e0 // 8) * 8
    k = e1 - ea
    ng = (k + 15) // 16
    nc2 = (k + CB2 - 1) // CB2

    pltpu.sync_copy(srcs.at[pl.ds(ea, KMAXS + 16)], srcb)
    pltpu.sync_copy(dsts.at[pl.ds(ea, KMAXS + 16)], dstb)

    e0v = jnp.full((16,), e0, jnp.int32)
    e1v = jnp.full((16,), e1, jnp.int32)
    n0v = jnp.full((16,), n0, jnp.int32)
    nbm1 = jnp.full((16,), NB - 1, jnp.int32)

    for h in range(H):
        pltpu.sync_copy(asrc_t.at[pl.ds(h * NPAD, NPAD)], asrc_v)
        pltpu.sync_copy(adst_t.at[pl.ds(h * NPAD + n0, NB)], adst_v)

        def zden(i, _):
            den_v[pl.ds(i * 16, 16)] = jnp.zeros((16,), jnp.float32)
            return 0
        lax.fori_loop(0, NB // 16, zden, 0)

        def logits(g):
            base = g * 16
            gev = jnp.full((16,), ea + base, jnp.int32) + ii
            active = (gev >= e0v) & (gev < e1v)
            sv = plsc.load_gather(srcb, [base + ii])
            dv = plsc.load_gather(dstb, [base + ii])
            ldst = jnp.minimum(jnp.maximum(dv - n0v, 0), nbm1)
            asv = plsc.load_gather(asrc_v, [jnp.minimum(sv, NPAD - 1)],
                                   mask=active)
            adv = plsc.load_gather(adst_v, [ldst], mask=active)
            a = asv + adv
            a = jnp.where(a > 0, a, 0.2 * a)
            ex = jnp.where(active, jnp.exp(a), 0.0)
            return dv, ldst, active, ex

        def p1(g, _):
            dv, ldst, active, ex = logits(g)
            cs = plsc.cumsum(ex)
            ndv = plsc.load_gather(dstb, [g * 16 + ii + 1])
            run_end = (dv != ndv) | (ii == 15)
            plsc.addupdate_scatter(den_v, [ldst], cs,
                                   mask=run_end & active)
            nldst = jnp.minimum(jnp.maximum(ndv - n0v, 0), nbm1)
            gev = jnp.full((16,), ea + g * 16, jnp.int32) + ii
            nact = (gev + 1 >= e0v) & (gev + 1 < e1v)
            plsc.addupdate_scatter(den_v, [nldst], -cs,
                                   mask=run_end & active & nact & (ii < 15))
            return 0
        lax.fori_loop(0, ng, p1, 0)

        def p2(g, _):
            dv, ldst, active, ex = logits(g)
            dd = plsc.load_gather(den_v, [ldst], mask=active)
            at = ex / jnp.where(active, dd, 1.0)
            plsc.store_scatter(attnb, [jnp.minimum(g * 16 + ii, KMAXS - 1)],
                               at, mask=active)
            return 0
        lax.fori_loop(0, ng, p2, 0)

        def zacc(i, _):
            for c in range(D // 16):
                acc[i, pl.ds(c * 16, 16)] = jnp.zeros((16,), jnp.float32)
            return 0
        lax.fori_loop(0, NB, zacc, 0)

        hoff = jnp.full((16,), h * NPAD, jnp.int32)

        def p3(ci, _):
            cbase = ci * CB2

            def mkidx(g, _):
                sv = plsc.load_gather(srcb, [cbase + g * 16 + ii])
                idxb[pl.ds(g * 16, 16)] = jnp.minimum(sv, NPAD - 1) + hoff
                return 0
            lax.fori_loop(0, CB2 // 16, mkidx, 0)
            pltpu.async_copy(h_t.at[idxb], rows_v, sem).wait()

            def edge(j, _):
                jv = jnp.full((16,), cbase + j, jnp.int32)
                gev = jnp.full((16,), ea + cbase + j, jnp.int32)
                active = (gev >= e0v) & (gev < e1v)
                av = plsc.load_gather(attnb,
                                      [jnp.minimum(jv, KMAXS - 1)],
                                      mask=active)
                dv = plsc.load_gather(dstb, [jv])
                ldst = jnp.minimum(jnp.maximum(dv - n0v, 0), nbm1)
                for c in range(D // 16):
                    rv = rows_v[j, pl.ds(c * 16, 16)]
                    plsc.addupdate_scatter(
                        acc, [ldst, c * 16 + ii], rv * av, mask=active)
                return 0
            lax.fori_loop(0, CB2, edge, 0)
            return 0
        lax.fori_loop(0, nc2, p3, 0)

        pltpu.sync_copy(
            acc, agg.at[pl.ds(n0, NB), pl.ds(h * D, D)])


def _edge_layer(H, D, h_t, asrc_t, adst_t, srcs, dsts, bnds):
    DR = h_t.shape[1]
    kern = pl.kernel(
        functools.partial(_edge_body, H, D, DR),
        out_type=jax.ShapeDtypeStruct((NPAD, H * D), jnp.float32),
        mesh=_mesh(),
        compiler_params=pltpu.CompilerParams(needs_layout_passes=False),
        scratch_types=[
            pltpu.VMEM((48,), jnp.int32),
            pltpu.VMEM((KMAXS + 16,), jnp.int32),
            pltpu.VMEM((KMAXS + 16,), jnp.int32),
            pltpu.VMEM((NPAD,), jnp.float32),
            pltpu.VMEM((NB,), jnp.float32),
            pltpu.VMEM((NB,), jnp.float32),
            pltpu.VMEM((KMAXS,), jnp.float32),
            pltpu.VMEM((CB2,), jnp.int32),
            pltpu.VMEM((CB2, DR), jnp.float32),
            pltpu.VMEM((NB, D), jnp.float32),
            pltpu.SemaphoreType.DMA,
        ],
    )
    return kern(h_t, asrc_t, adst_t, srcs, dsts, bnds)


def kernel(x, edge_index, W1, a_src1, a_dst1, b1, W2, a_src2, a_dst2, b2,
           W3, a_src3, a_dst3, b3, Wr, br):
    src = edge_index[0].astype(jnp.int32)
    dst = edge_index[1].astype(jnp.int32)
    order = jnp.argsort(dst)
    srcs = jnp.zeros((EPAD,), jnp.int32).at[:E].set(src[order])
    dsts = jnp.full((EPAD,), NPAD + 7, jnp.int32).at[:E].set(dst[order])
    starts = jnp.arange(NW + 1, dtype=jnp.int32) * NB
    bnds = jnp.searchsorted(dsts[:E], starts).astype(jnp.int32)
    bnds = jnp.concatenate([bnds, jnp.full((48 - NW - 1,), E, jnp.int32)])

    xp = jnp.zeros((NPAD, D_IN), jnp.float32).at[:N].set(x)

    h1, as1, ad1 = _dense_layer(xp, jnp.zeros((D_IN,), jnp.float32), W1,
                                a_src1, a_dst1, HEADS, HID, False)
    agg1 = _edge_layer(HEADS, HID, h1.reshape(HEADS * NPAD, HID),
                       as1.reshape(HEADS * NPAD), ad1.reshape(HEADS * NPAD),
                       srcs, dsts, bnds)

    h2, as2, ad2 = _dense_layer(agg1, b1, W2, a_src2, a_dst2, 1, HID, True)
    agg2 = _edge_layer(1, HID, h2.reshape(NPAD, HID),
                       as2.reshape(NPAD), ad2.reshape(NPAD),
                       srcs, dsts, bnds)

    h3, as3, ad3 = _dense_layer(agg2, b2, W3, a_src3, a_dst3, 1, HID // 2,
                                True)
    h3p = jnp.pad(h3.reshape(NPAD, HID // 2), ((0, 0), (0, HID // 2)))
    agg3 = _edge_layer(1, HID // 2, h3p,
                       as3.reshape(NPAD), ad3.reshape(NPAD),
                       srcs, dsts, bnds)

    y = _final_layer(agg3, b3, Wr, br)
    return y[:N]


# speedup vs baseline: 10.3798x; 1.0284x over previous
import functools

import jax
import jax.numpy as jnp
from jax import lax
from jax.experimental import pallas as pl
from jax.experimental.pallas import tpu as pltpu
from jax.experimental.pallas import tpu_sc as plsc

N = 10000
E = 320000
D_IN = 128
HID = 128
HEADS = 8

NW = 32
NB = 320
NPAD = NW * NB
BN = 512
NBLK = NPAD // BN
KMAXS = 11008
CB2 = 256
EPAD = E + KMAXS + 32

def _mesh():
    return plsc.VectorSubcoreMesh(core_axis_name="c", subcore_axis_name="s")


def _dense_body(apply_elu, x_ref, b_ref, W_ref, a_s_ref, a_d_ref,
                h_ref, as_ref, ad_ref):
    xb = x_ref[...]
    if apply_elu:
        xb = xb + b_ref[0]
        xb = jnp.where(xb > 0, xb, jnp.exp(jnp.minimum(xb, 0.0)) - 1.0)
    h = jnp.dot(xb, W_ref[...], preferred_element_type=jnp.float32)
    h_ref[0] = h
    as_ref[0, 0] = jnp.sum(h * a_s_ref[0], axis=1)
    ad_ref[0, 0] = jnp.sum(h * a_d_ref[0], axis=1)


def _dense_layer(x, b_prev, W, a_s, a_d, heads, dout, apply_elu):
    din = x.shape[1]
    grid = (NBLK, heads)
    out_shapes = [
        jax.ShapeDtypeStruct((heads, NPAD, dout), jnp.float32),
        jax.ShapeDtypeStruct((heads, 1, NPAD), jnp.float32),
        jax.ShapeDtypeStruct((heads, 1, NPAD), jnp.float32),
    ]
    fn = pl.pallas_call(
        functools.partial(_dense_body, apply_elu),
        grid=grid,
        in_specs=[
            pl.BlockSpec((BN, din), lambda i, h: (i, 0)),
            pl.BlockSpec((1, din), lambda i, h: (0, 0)),
            pl.BlockSpec((din, dout), lambda i, h: (0, h)),
            pl.BlockSpec((1, 1, dout), lambda i, h: (h, 0, 0)),
            pl.BlockSpec((1, 1, dout), lambda i, h: (h, 0, 0)),
        ],
        out_specs=[
            pl.BlockSpec((1, BN, dout), lambda i, h: (h, i, 0)),
            pl.BlockSpec((1, 1, BN), lambda i, h: (h, 0, i)),
            pl.BlockSpec((1, 1, BN), lambda i, h: (h, 0, i)),
        ],
        out_shape=out_shapes,
    )
    return fn(x, b_prev.reshape(1, din), W, a_s.reshape(heads, 1, dout),
              a_d.reshape(heads, 1, dout))


def _final_body(x_ref, b_ref, w_ref, br_ref, y_ref):
    xb = x_ref[...] + b_ref[0]
    xb = jnp.where(xb > 0, xb, jnp.exp(jnp.minimum(xb, 0.0)) - 1.0)
    y = jnp.sum(xb * w_ref[0], axis=1) + br_ref[0, 0]
    y_ref[0, 0] = jax.nn.sigmoid(y)


def _final_layer(x, b_prev, Wr, br):
    din = x.shape[1]
    fn = pl.pallas_call(
        _final_body,
        grid=(NBLK,),
        in_specs=[
            pl.BlockSpec((BN, din), lambda i: (i, 0)),
            pl.BlockSpec((1, din), lambda i: (0, 0)),
            pl.BlockSpec((1, din), lambda i: (0, 0)),
            pl.BlockSpec((1, 1), lambda i: (0, 0)),
        ],
        out_specs=pl.BlockSpec((1, 1, BN), lambda i: (i, 0, 0)),
        out_shape=jax.ShapeDtypeStruct((NBLK, 1, BN), jnp.float32),
    )
    y = fn(x, b_prev.reshape(1, din), Wr.reshape(1, din),
           br.reshape(1, 1))
    return y.reshape(NPAD)


def _edge_body(H, D, DR, h_t, asrc_t, adst_t, srcs, dsts, bnds, agg,
               bv, srcb, dstb, asrc_v, adst_v, den_v, attnb, idxb,
               rows_v, acc, sem):
    wid = lax.axis_index("s") * 2 + lax.axis_index("c")
    n0 = wid * NB
    pltpu.sync_copy(bnds, bv)
    ii = lax.iota(jnp.int32, 16)

    def _bnd(w):
        chunk = bv[pl.ds((w // 16) * 16, 16)]
        return jnp.sum(jnp.where(ii == w % 16, chunk, 0))

    e0 = _bnd(wid)
    e1 = _bnd(wid + 1)
    ea = (e0 // 8) * 8
    k = e1 - ea
    ng = (k + 15) // 16
    nc2 = (k + CB2 - 1) // CB2

    pltpu.sync_copy(srcs.at[pl.ds(ea, KMAXS + 16)], srcb)
    pltpu.sync_copy(dsts.at[pl.ds(ea, KMAXS + 16)], dstb)

    e0v = jnp.full((16,), e0, jnp.int32)
    e1v = jnp.full((16,), e1, jnp.int32)
    n0v = jnp.full((16,), n0, jnp.int32)
    nbm1 = jnp.full((16,), NB - 1, jnp.int32)
    nbv = jnp.full((16,), NB, jnp.int32)

    def enc(g, _):
        base = g * 16
        dv = dstb[pl.ds(base, 16)]
        gev = jnp.full((16,), ea + base, jnp.int32) + ii
        active = (gev >= e0v) & (gev < e1v)
        ldst = jnp.minimum(jnp.maximum(dv - n0v, 0), nbm1)
        dstb[pl.ds(base, 16)] = jnp.where(active, ldst, nbv)
        return 0
    lax.fori_loop(0, KMAXS // 16, enc, 0)

    for h in range(H):
        pltpu.sync_copy(asrc_t.at[pl.ds(h * NPAD, NPAD)], asrc_v)
        pltpu.sync_copy(adst_t.at[pl.ds(h * NPAD + n0, NB)],
                        adst_v.at[pl.ds(0, NB)])

        def zden(i, _):
            den_v[pl.ds(i * 16, 16)] = jnp.zeros((16,), jnp.float32)
            return 0
        lax.fori_loop(0, (NB + 16) // 16, zden, 0)

        def logits(g):
            base = g * 16
            ldst = dstb[pl.ds(base, 16)]
            active = ldst < nbv
            sv = plsc.load_gather(srcb, [base + ii])
            asv = plsc.load_gather(asrc_v, [jnp.minimum(sv, NPAD - 1)])
            adv = plsc.load_gather(adst_v, [ldst])
            a = asv + adv
            a = jnp.where(a > 0, a, 0.2 * a)
            ex = jnp.where(active, jnp.exp(a), 0.0)
            return ldst, active, ex

        def p1(g, _):
            ldst, active, ex = logits(g)
            cs = plsc.cumsum(ex)
            nldst = plsc.load_gather(dstb, [g * 16 + ii + 1])
            run_end = (ldst != nldst) | (ii == 15)
            plsc.addupdate_scatter(den_v, [ldst], cs,
                                   mask=run_end & active)
            plsc.addupdate_scatter(den_v, [nldst], -cs,
                                   mask=run_end & active & (nldst < nbv)
                                   & (ii < 15))
            return 0
        lax.fori_loop(0, ng, p1, 0)

        def p2(g, _):
            ldst, active, ex = logits(g)
            dd = plsc.load_gather(den_v, [ldst])
            attnb[pl.ds(g * 16, 16)] = ex / jnp.where(active, dd, 1.0)
            return 0
        lax.fori_loop(0, ng, p2, 0)

        def zacc(i, _):
            for c in range(D // 16):
                acc[i, pl.ds(c * 16, 16)] = jnp.zeros((16,), jnp.float32)
            return 0
        lax.fori_loop(0, NB + 8, zacc, 0)

        hoff = jnp.full((16,), h * NPAD, jnp.int32)

        def p3(ci, _):
            cbase = ci * CB2

            def mkidx(g, _):
                sv = plsc.load_gather(srcb, [cbase + g * 16 + ii])
                idxb[pl.ds(g * 16, 16)] = jnp.minimum(sv, NPAD - 1) + hoff
                return 0
            lax.fori_loop(0, CB2 // 16, mkidx, 0)
            pltpu.async_copy(h_t.at[idxb], rows_v, sem).wait()

            def edge(j2, _):
                for u in range(2):
                    j = j2 * 2 + u
                    jv = jnp.full((16,), cbase + j, jnp.int32)
                    av = plsc.load_gather(attnb, [jv])
                    lv = plsc.load_gather(dstb, [jv])
                    for c in range(D // 16):
                        rv = rows_v[j, pl.ds(c * 16, 16)]
                        plsc.addupdate_scatter(
                            acc, [lv, c * 16 + ii], rv * av)
                return 0
            lax.fori_loop(0, CB2 // 2, edge, 0)
            return 0
        lax.fori_loop(0, nc2, p3, 0)

        pltpu.sync_copy(
            acc.at[pl.ds(0, NB)], agg.at[pl.ds(n0, NB), pl.ds(h * D, D)])


def _edge_layer(H, D, h_t, asrc_t, adst_t, srcs, dsts, bnds):
    DR = h_t.shape[1]
    kern = pl.kernel(
        functools.partial(_edge_body, H, D, DR),
        out_type=jax.ShapeDtypeStruct((NPAD, H * D), jnp.float32),
        mesh=_mesh(),
        compiler_params=pltpu.CompilerParams(needs_layout_passes=False),
        scratch_types=[
            pltpu.VMEM((48,), jnp.int32),
            pltpu.VMEM((KMAXS + 16,), jnp.int32),
            pltpu.VMEM((KMAXS + 16,), jnp.int32),
            pltpu.VMEM((NPAD,), jnp.float32),
            pltpu.VMEM((NB + 16,), jnp.float32),
            pltpu.VMEM((NB + 16,), jnp.float32),
            pltpu.VMEM((KMAXS,), jnp.float32),
            pltpu.VMEM((CB2,), jnp.int32),
            pltpu.VMEM((CB2, DR), jnp.float32),
            pltpu.VMEM((NB + 8, D), jnp.float32),
            pltpu.SemaphoreType.DMA,
        ],
    )
    return kern(h_t, asrc_t, adst_t, srcs, dsts, bnds)


def kernel(x, edge_index, W1, a_src1, a_dst1, b1, W2, a_src2, a_dst2, b2,
           W3, a_src3, a_dst3, b3, Wr, br):
    src = edge_index[0].astype(jnp.int32)
    dst = edge_index[1].astype(jnp.int32)
    order = jnp.argsort(dst)
    srcs = jnp.zeros((EPAD,), jnp.int32).at[:E].set(src[order])
    dsts = jnp.full((EPAD,), NPAD + 7, jnp.int32).at[:E].set(dst[order])
    starts = jnp.arange(NW + 1, dtype=jnp.int32) * NB
    bnds = jnp.searchsorted(dsts[:E], starts).astype(jnp.int32)
    bnds = jnp.concatenate([bnds, jnp.full((48 - NW - 1,), E, jnp.int32)])

    xp = jnp.zeros((NPAD, D_IN), jnp.float32).at[:N].set(x)

    h1, as1, ad1 = _dense_layer(xp, jnp.zeros((D_IN,), jnp.float32), W1,
                                a_src1, a_dst1, HEADS, HID, False)
    agg1 = _edge_layer(HEADS, HID, h1.reshape(HEADS * NPAD, HID),
                       as1.reshape(HEADS * NPAD), ad1.reshape(HEADS * NPAD),
                       srcs, dsts, bnds)

    h2, as2, ad2 = _dense_layer(agg1, b1, W2, a_src2, a_dst2, 1, HID, True)
    agg2 = _edge_layer(1, HID, h2.reshape(NPAD, HID),
                       as2.reshape(NPAD), ad2.reshape(NPAD),
                       srcs, dsts, bnds)

    h3, as3, ad3 = _dense_layer(agg2, b2, W3, a_src3, a_dst3, 1, HID // 2,
                                True)
    h3p = jnp.pad(h3.reshape(NPAD, HID // 2), ((0, 0), (0, HID // 2)))
    agg3 = _edge_layer(1, HID // 2, h3p,
                       as3.reshape(NPAD), ad3.reshape(NPAD),
                       srcs, dsts, bnds)

    y = _final_layer(agg3, b3, Wr, br)
    return y[:N]


# speedup vs baseline: 22.3556x; 2.1538x over previous
import functools

import jax
import jax.numpy as jnp
from jax import lax
from jax.experimental import pallas as pl
from jax.experimental.pallas import tpu as pltpu
from jax.experimental.pallas import tpu_sc as plsc

N = 10000
E = 320000
D_IN = 128
HID = 128
HEADS = 8

NW = 32
NB = 320
NPAD = NW * NB
BN = 512
NBLK = NPAD // BN
KMAXS = 11008
CB2 = 256
EPAD = E + KMAXS + 160

def _mesh():
    return plsc.VectorSubcoreMesh(core_axis_name="c", subcore_axis_name="s")


def _dense_body(apply_elu, x_ref, b_ref, W_ref, a_s_ref, a_d_ref,
                h_ref, as_ref, ad_ref):
    xb = x_ref[...]
    if apply_elu:
        xb = xb + b_ref[0]
        xb = jnp.where(xb > 0, xb, jnp.exp(jnp.minimum(xb, 0.0)) - 1.0)
    h = jnp.dot(xb, W_ref[...], preferred_element_type=jnp.float32)
    h_ref[0] = h
    as_ref[0, 0] = jnp.sum(h * a_s_ref[0], axis=1)
    ad_ref[0, 0] = jnp.sum(h * a_d_ref[0], axis=1)


def _dense_layer(x, b_prev, W, a_s, a_d, heads, dout, apply_elu):
    din = x.shape[1]
    grid = (NBLK, heads)
    out_shapes = [
        jax.ShapeDtypeStruct((heads, NPAD, dout), jnp.float32),
        jax.ShapeDtypeStruct((heads, 1, NPAD), jnp.float32),
        jax.ShapeDtypeStruct((heads, 1, NPAD), jnp.float32),
    ]
    fn = pl.pallas_call(
        functools.partial(_dense_body, apply_elu),
        grid=grid,
        in_specs=[
            pl.BlockSpec((BN, din), lambda i, h: (i, 0)),
            pl.BlockSpec((1, din), lambda i, h: (0, 0)),
            pl.BlockSpec((din, dout), lambda i, h: (0, h)),
            pl.BlockSpec((1, 1, dout), lambda i, h: (h, 0, 0)),
            pl.BlockSpec((1, 1, dout), lambda i, h: (h, 0, 0)),
        ],
        out_specs=[
            pl.BlockSpec((1, BN, dout), lambda i, h: (h, i, 0)),
            pl.BlockSpec((1, 1, BN), lambda i, h: (h, 0, i)),
            pl.BlockSpec((1, 1, BN), lambda i, h: (h, 0, i)),
        ],
        out_shape=out_shapes,
    )
    return fn(x, b_prev.reshape(1, din), W, a_s.reshape(heads, 1, dout),
              a_d.reshape(heads, 1, dout))


def _final_body(x_ref, b_ref, w_ref, br_ref, y_ref):
    xb = x_ref[...] + b_ref[0]
    xb = jnp.where(xb > 0, xb, jnp.exp(jnp.minimum(xb, 0.0)) - 1.0)
    y = jnp.sum(xb * w_ref[0], axis=1) + br_ref[0, 0]
    y_ref[0, 0] = jax.nn.sigmoid(y)


def _final_layer(x, b_prev, Wr, br):
    din = x.shape[1]
    fn = pl.pallas_call(
        _final_body,
        grid=(NBLK,),
        in_specs=[
            pl.BlockSpec((BN, din), lambda i: (i, 0)),
            pl.BlockSpec((1, din), lambda i: (0, 0)),
            pl.BlockSpec((1, din), lambda i: (0, 0)),
            pl.BlockSpec((1, 1), lambda i: (0, 0)),
        ],
        out_specs=pl.BlockSpec((1, 1, BN), lambda i: (i, 0, 0)),
        out_shape=jax.ShapeDtypeStruct((NBLK, 1, BN), jnp.float32),
    )
    y = fn(x, b_prev.reshape(1, din), Wr.reshape(1, din),
           br.reshape(1, 1))
    return y.reshape(NPAD)


CB = 112
SROWS = NB + 8


def _edge_body(H, D, DR, h_t, asrc_t, adst_t, srcs, dsts, bnds, agg,
               bv, srcb, dstb, asrc_v, adst_v, den_v, attnb,
               ig0, ig1, ig2, is0, is1, is2, r0, r1, r2, sacc,
               sg0, sg1, sg2, ss0, ss1, ss2):
    igs = (ig0, ig1, ig2)
    iss = (is0, is1, is2)
    rows = (r0, r1, r2)
    sgs = (sg0, sg1, sg2)
    sss = (ss0, ss1, ss2)
    wid = lax.axis_index("s") * 2 + lax.axis_index("c")
    sid = lax.axis_index("s")
    n0 = wid * NB
    pltpu.sync_copy(bnds, bv)
    ii = lax.iota(jnp.int32, 16)

    def _bnd(w):
        chunk = bv[pl.ds((w // 16) * 16, 16)]
        return jnp.sum(jnp.where(ii == w % 16, chunk, 0))

    e0 = _bnd(wid)
    e1 = _bnd(wid + 1)
    ea = (e0 // 8) * 8
    k = e1 - ea
    ng = (k + 15) // 16
    nc2 = (k + CB - 1) // CB

    pltpu.sync_copy(srcs.at[pl.ds(ea, KMAXS + 128)], srcb)
    pltpu.sync_copy(dsts.at[pl.ds(ea, KMAXS + 128)], dstb)

    e0v = jnp.full((16,), e0, jnp.int32)
    e1v = jnp.full((16,), e1, jnp.int32)
    n0v = jnp.full((16,), n0, jnp.int32)
    nbm1 = jnp.full((16,), NB - 1, jnp.int32)
    nbv = jnp.full((16,), NB, jnp.int32)
    soff = jnp.full((16,), sid * SROWS, jnp.int32)

    def enc(g, _):
        base = g * 16
        dv = dstb[pl.ds(base, 16)]
        gev = jnp.full((16,), ea + base, jnp.int32) + ii
        active = (gev >= e0v) & (gev < e1v)
        ldst = jnp.minimum(jnp.maximum(dv - n0v, 0), nbm1)
        dstb[pl.ds(base, 16)] = jnp.where(active, ldst, nbv)
        return 0
    lax.fori_loop(0, (KMAXS + 16) // 16, enc, 0)

    for h in range(H):
        pltpu.sync_copy(asrc_t.at[pl.ds(h * NPAD, NPAD)], asrc_v)
        pltpu.sync_copy(adst_t.at[pl.ds(h * NPAD + n0, NB)],
                        adst_v.at[pl.ds(0, NB)])

        def zden(i, _):
            den_v[pl.ds(i * 16, 16)] = jnp.zeros((16,), jnp.float32)
            return 0
        lax.fori_loop(0, (NB + 16) // 16, zden, 0)

        def logits(g):
            base = g * 16
            ldst = dstb[pl.ds(base, 16)]
            active = ldst < nbv
            sv = plsc.load_gather(srcb, [base + ii])
            asv = plsc.load_gather(asrc_v, [jnp.minimum(sv, NPAD - 1)])
            adv = plsc.load_gather(adst_v, [ldst])
            a = asv + adv
            a = jnp.where(a > 0, a, 0.2 * a)
            ex = jnp.where(active, jnp.exp(a), 0.0)
            return ldst, active, ex

        def p1(g, _):
            ldst, active, ex = logits(g)
            cs = plsc.cumsum(ex)
            nldst = plsc.load_gather(dstb, [g * 16 + ii + 1])
            run_end = (ldst != nldst) | (ii == 15)
            plsc.addupdate_scatter(den_v, [ldst], cs,
                                   mask=run_end & active)
            plsc.addupdate_scatter(den_v, [nldst], -cs,
                                   mask=run_end & active & (nldst < nbv)
                                   & (ii < 15))
            return 0
        lax.fori_loop(0, ng, p1, 0)

        def p2(g, _):
            ldst, active, ex = logits(g)
            dd = plsc.load_gather(den_v, [ldst])
            attnb[pl.ds(g * 16, 16)] = ex / jnp.where(active, dd, 1.0)
            return 0
        lax.fori_loop(0, ng, p2, 0)

        def zr(i, _):
            for c in range(DR // 16):
                r0[i, pl.ds(c * 16, 16)] = jnp.zeros((16,), jnp.float32)
            return 0
        lax.fori_loop(0, CB, zr, 0)
        pltpu.sync_copy(r0, sacc.at[pl.ds(sid * SROWS, CB)])
        pltpu.sync_copy(r0, sacc.at[pl.ds(sid * SROWS + CB, CB)])
        pltpu.sync_copy(r0.at[pl.ds(0, SROWS - 2 * CB)],
                        sacc.at[pl.ds(sid * SROWS + 2 * CB,
                                      SROWS - 2 * CB)])

        hoff = jnp.full((16,), h * NPAD, jnp.int32)

        def mkidx(ci, b):
            cbase = ci * CB
            def g1(g, _):
                sv = plsc.load_gather(srcb, [cbase + g * 16 + ii])
                igs[b][pl.ds(g * 16, 16)] = (
                    jnp.minimum(sv, NPAD - 1) + hoff)
                lv = jnp.minimum(dstb[pl.ds(cbase + g * 16, 16)], nbv)
                iss[b][pl.ds(g * 16, 16)] = lv + soff
                return 0
            lax.fori_loop(0, CB // 16, g1, 0)

        for p in range(2):
            mkidx(p, p)
            pltpu.async_copy(h_t.at[igs[p]], rows[p], sgs[p])

        def p3(ci3, _):
            for b3 in range(3):
                ci = ci3 * 3 + b3
                b = b3

                @pl.when(ci < nc2)
                def _():
                    nb_ = (b3 + 2) % 3

                    @pl.when((ci >= 1) & (ci + 2 < nc2))
                    def _():
                        pltpu.make_async_copy(
                            rows[nb_], sacc.at[iss[nb_]],
                            sss[nb_]).wait()

                    @pl.when(ci + 2 < nc2)
                    def _():
                        mkidx(ci + 2, nb_)
                        pltpu.async_copy(h_t.at[igs[nb_]], rows[nb_],
                                         sgs[nb_])
                    pltpu.make_async_copy(h_t.at[igs[b]], rows[b],
                                          sgs[b]).wait()

                    def edge(j2, _):
                        for u in range(2):
                            j = j2 * 2 + u
                            jv = jnp.full((16,), ci * CB + j, jnp.int32)
                            av = plsc.load_gather(attnb, [jv])
                            for c in range(D // 16):
                                rv = rows[b][j, pl.ds(c * 16, 16)]
                                rows[b][j, pl.ds(c * 16, 16)] = rv * av
                        return 0
                    lax.fori_loop(0, CB // 2, edge, 0)
                    pltpu.async_copy(rows[b], sacc.at[iss[b]], sss[b],
                                     add=True)
            return 0
        lax.fori_loop(0, (nc2 + 2) // 3, p3, 0)

        for b in range(3):
            pltpu.make_async_copy(rows[b], sacc.at[iss[b]],
                                  sss[b]).wait()

        for seg, ln in ((0, CB), (1, CB), (2, NB - 2 * CB)):
            pltpu.sync_copy(
                sacc.at[pl.ds(sid * SROWS + seg * CB, ln)],
                r1.at[pl.ds(0, ln)])
            pltpu.sync_copy(
                r1.at[pl.ds(0, ln)],
                agg.at[pl.ds(n0 + seg * CB, ln), pl.ds(h * DR, DR)])


def _edge_layer(H, D, h_t, asrc_t, adst_t, srcs, dsts, bnds):
    DR = h_t.shape[1]
    kern = pl.kernel(
        functools.partial(_edge_body, H, D, DR),
        out_type=jax.ShapeDtypeStruct((NPAD, H * DR), jnp.float32),
        mesh=_mesh(),
        compiler_params=pltpu.CompilerParams(needs_layout_passes=False),
        scratch_types=[
            pltpu.VMEM((48,), jnp.int32),
            pltpu.VMEM((KMAXS + 128,), jnp.int32),
            pltpu.VMEM((KMAXS + 128,), jnp.int32),
            pltpu.VMEM((NPAD,), jnp.float32),
            pltpu.VMEM((NB + 16,), jnp.float32),
            pltpu.VMEM((NB + 16,), jnp.float32),
            pltpu.VMEM((KMAXS + 128,), jnp.float32),
            pltpu.VMEM((CB,), jnp.int32),
            pltpu.VMEM((CB,), jnp.int32),
            pltpu.VMEM((CB,), jnp.int32),
            pltpu.VMEM((CB,), jnp.int32),
            pltpu.VMEM((CB,), jnp.int32),
            pltpu.VMEM((CB,), jnp.int32),
            pltpu.VMEM((CB, DR), jnp.float32),
            pltpu.VMEM((CB, DR), jnp.float32),
            pltpu.VMEM((CB, DR), jnp.float32),
            pltpu.VMEM_SHARED((16 * SROWS, DR), jnp.float32),
            pltpu.SemaphoreType.DMA,
            pltpu.SemaphoreType.DMA,
            pltpu.SemaphoreType.DMA,
            pltpu.SemaphoreType.DMA,
            pltpu.SemaphoreType.DMA,
            pltpu.SemaphoreType.DMA,
        ],
    )
    return kern(h_t, asrc_t, adst_t, srcs, dsts, bnds)


def kernel(x, edge_index, W1, a_src1, a_dst1, b1, W2, a_src2, a_dst2, b2,
           W3, a_src3, a_dst3, b3, Wr, br):
    src = edge_index[0].astype(jnp.int32)
    dst = edge_index[1].astype(jnp.int32)
    order = jnp.argsort(dst)
    srcs = jnp.zeros((EPAD,), jnp.int32).at[:E].set(src[order])
    dsts = jnp.full((EPAD,), NPAD + 7, jnp.int32).at[:E].set(dst[order])
    starts = jnp.arange(NW + 1, dtype=jnp.int32) * NB
    bnds = jnp.searchsorted(dsts[:E], starts).astype(jnp.int32)
    bnds = jnp.concatenate([bnds, jnp.full((48 - NW - 1,), E, jnp.int32)])

    xp = jnp.zeros((NPAD, D_IN), jnp.float32).at[:N].set(x)

    h1, as1, ad1 = _dense_layer(xp, jnp.zeros((D_IN,), jnp.float32), W1,
                                a_src1, a_dst1, HEADS, HID, False)
    agg1 = _edge_layer(HEADS, HID, h1.reshape(HEADS * NPAD, HID),
                       as1.reshape(HEADS * NPAD), ad1.reshape(HEADS * NPAD),
                       srcs, dsts, bnds)

    h2, as2, ad2 = _dense_layer(agg1, b1, W2, a_src2, a_dst2, 1, HID, True)
    agg2 = _edge_layer(1, HID, h2.reshape(NPAD, HID),
                       as2.reshape(NPAD), ad2.reshape(NPAD),
                       srcs, dsts, bnds)

    h3, as3, ad3 = _dense_layer(agg2, b2, W3, a_src3, a_dst3, 1, HID // 2,
                                True)
    h3p = jnp.pad(h3.reshape(NPAD, HID // 2), ((0, 0), (0, HID // 2)))
    agg3 = _edge_layer(1, HID // 2, h3p,
                       as3.reshape(NPAD), ad3.reshape(NPAD),
                       srcs, dsts, bnds)

    agg3 = agg3[:, :HID // 2]

    y = _final_layer(agg3, b3, Wr, br)
    return y[:N]


# speedup vs baseline: 22.5849x; 1.0103x over previous
import functools

import jax
import jax.numpy as jnp
from jax import lax
from jax.experimental import pallas as pl
from jax.experimental.pallas import tpu as pltpu
from jax.experimental.pallas import tpu_sc as plsc

N = 10000
E = 320000
D_IN = 128
HID = 128
HEADS = 8

NW = 32
NB = 320
NPAD = NW * NB
BN = 512
NBLK = NPAD // BN
KMAXS = 11136
CB2 = 256
EPAD = E + KMAXS + 160

def _mesh():
    return plsc.VectorSubcoreMesh(core_axis_name="c", subcore_axis_name="s")


def _dense_body(apply_elu, x_ref, b_ref, W_ref, a_s_ref, a_d_ref,
                h_ref, as_ref, ad_ref):
    xb = x_ref[...]
    if apply_elu:
        xb = xb + b_ref[0]
        xb = jnp.where(xb > 0, xb, jnp.exp(jnp.minimum(xb, 0.0)) - 1.0)
    h = jnp.dot(xb, W_ref[...], preferred_element_type=jnp.float32)
    h_ref[0] = h
    as_ref[0, 0] = jnp.sum(h * a_s_ref[0], axis=1)
    ad_ref[0, 0] = jnp.sum(h * a_d_ref[0], axis=1)


def _dense_layer(x, b_prev, W, a_s, a_d, heads, dout, apply_elu):
    din = x.shape[1]
    grid = (NBLK, heads)
    out_shapes = [
        jax.ShapeDtypeStruct((heads, NPAD, dout), jnp.float32),
        jax.ShapeDtypeStruct((heads, 1, NPAD), jnp.float32),
        jax.ShapeDtypeStruct((heads, 1, NPAD), jnp.float32),
    ]
    fn = pl.pallas_call(
        functools.partial(_dense_body, apply_elu),
        grid=grid,
        in_specs=[
            pl.BlockSpec((BN, din), lambda i, h: (i, 0)),
            pl.BlockSpec((1, din), lambda i, h: (0, 0)),
            pl.BlockSpec((din, dout), lambda i, h: (0, h)),
            pl.BlockSpec((1, 1, dout), lambda i, h: (h, 0, 0)),
            pl.BlockSpec((1, 1, dout), lambda i, h: (h, 0, 0)),
        ],
        out_specs=[
            pl.BlockSpec((1, BN, dout), lambda i, h: (h, i, 0)),
            pl.BlockSpec((1, 1, BN), lambda i, h: (h, 0, i)),
            pl.BlockSpec((1, 1, BN), lambda i, h: (h, 0, i)),
        ],
        out_shape=out_shapes,
    )
    return fn(x, b_prev.reshape(1, din), W, a_s.reshape(heads, 1, dout),
              a_d.reshape(heads, 1, dout))


def _final_body(x_ref, b_ref, w_ref, br_ref, y_ref):
    xb = x_ref[...] + b_ref[0]
    xb = jnp.where(xb > 0, xb, jnp.exp(jnp.minimum(xb, 0.0)) - 1.0)
    y = jnp.sum(xb * w_ref[0], axis=1) + br_ref[0, 0]
    y_ref[0, 0] = jax.nn.sigmoid(y)


def _final_layer(x, b_prev, Wr, br):
    din = x.shape[1]
    fn = pl.pallas_call(
        _final_body,
        grid=(NBLK,),
        in_specs=[
            pl.BlockSpec((BN, din), lambda i: (i, 0)),
            pl.BlockSpec((1, din), lambda i: (0, 0)),
            pl.BlockSpec((1, din), lambda i: (0, 0)),
            pl.BlockSpec((1, 1), lambda i: (0, 0)),
        ],
        out_specs=pl.BlockSpec((1, 1, BN), lambda i: (i, 0, 0)),
        out_shape=jax.ShapeDtypeStruct((NBLK, 1, BN), jnp.float32),
    )
    y = fn(x, b_prev.reshape(1, din), Wr.reshape(1, din),
           br.reshape(1, 1))
    return y.reshape(NPAD)


CB = 112
SROWS = NB + 8


def _edge_body(H, D, DR, h_t, asrc_t, adst_t, srcs, dsts, bnds, agg,
               bv, srcb, dstb, asrc_v, adst_v, den_v, attnb,
               ig0, ig1, ig2, is0, is1, is2, r0, r1, r2, sacc,
               sg0, sg1, sg2, ss0, ss1, ss2):
    igs = (ig0, ig1, ig2)
    iss = (is0, is1, is2)
    rows = (r0, r1, r2)
    sgs = (sg0, sg1, sg2)
    sss = (ss0, ss1, ss2)
    wid = lax.axis_index("s") * 2 + lax.axis_index("c")
    sid = lax.axis_index("s")
    n0 = wid * NB
    pltpu.sync_copy(bnds, bv)
    ii = lax.iota(jnp.int32, 16)

    def _bnd(w):
        chunk = bv[pl.ds((w // 16) * 16, 16)]
        return jnp.sum(jnp.where(ii == w % 16, chunk, 0))

    e0 = _bnd(wid)
    e1 = _bnd(wid + 1)
    ea = (e0 // 8) * 8
    k = e1 - ea
    ng = (k + 15) // 16
    nc2 = (k + CB - 1) // CB

    pltpu.sync_copy(srcs.at[pl.ds(ea, KMAXS + 128)], srcb)
    pltpu.sync_copy(dsts.at[pl.ds(ea, KMAXS + 128)], dstb)

    e0v = jnp.full((16,), e0, jnp.int32)
    e1v = jnp.full((16,), e1, jnp.int32)
    n0v = jnp.full((16,), n0, jnp.int32)
    nbm1 = jnp.full((16,), NB - 1, jnp.int32)
    nbv = jnp.full((16,), NB, jnp.int32)
    soff = jnp.full((16,), sid * SROWS, jnp.int32)

    def enc(g, _):
        base = g * 16
        dv = dstb[pl.ds(base, 16)]
        gev = jnp.full((16,), ea + base, jnp.int32) + ii
        active = (gev >= e0v) & (gev < e1v)
        ldst = jnp.minimum(jnp.maximum(dv - n0v, 0), nbm1)
        dstb[pl.ds(base, 16)] = jnp.where(active, ldst, nbv)
        return 0
    lax.fori_loop(0, (KMAXS + 16) // 16, enc, 0)

    for h in range(H):
        pltpu.sync_copy(asrc_t.at[pl.ds(h * NPAD, NPAD)], asrc_v)
        pltpu.sync_copy(adst_t.at[pl.ds(h * NPAD + n0, NB)],
                        adst_v.at[pl.ds(0, NB)])

        def zden(i, _):
            den_v[pl.ds(i * 16, 16)] = jnp.zeros((16,), jnp.float32)
            return 0
        lax.fori_loop(0, (NB + 16) // 16, zden, 0)

        def logits(g):
            base = g * 16
            ldst = dstb[pl.ds(base, 16)]
            active = ldst < nbv
            sv = plsc.load_gather(srcb, [base + ii])
            asv = plsc.load_gather(asrc_v, [jnp.minimum(sv, NPAD - 1)])
            adv = plsc.load_gather(adst_v, [ldst])
            a = asv + adv
            a = jnp.where(a > 0, a, 0.2 * a)
            ex = jnp.where(active, jnp.exp(a), 0.0)
            return ldst, active, ex

        def p1(g, _):
            ldst, active, ex = logits(g)
            attnb[pl.ds(g * 16, 16)] = ex
            cs = plsc.cumsum(ex)
            nldst = plsc.load_gather(dstb, [g * 16 + ii + 1])
            run_end = (ldst != nldst) | (ii == 15)
            plsc.addupdate_scatter(den_v, [ldst], cs,
                                   mask=run_end & active)
            plsc.addupdate_scatter(den_v, [nldst], -cs,
                                   mask=run_end & active & (nldst < nbv)
                                   & (ii < 15))
            return 0
        lax.fori_loop(0, ng, p1, 0)

        def p2(g, _):
            base = g * 16
            ldst = dstb[pl.ds(base, 16)]
            ex = attnb[pl.ds(base, 16)]
            dd = plsc.load_gather(den_v, [ldst])
            attnb[pl.ds(base, 16)] = ex / jnp.where(ldst < nbv, dd, 1.0)
            return 0
        lax.fori_loop(0, ng, p2, 0)

        def zr(i, _):
            for c in range(DR // 16):
                r0[i, pl.ds(c * 16, 16)] = jnp.zeros((16,), jnp.float32)
            return 0
        lax.fori_loop(0, CB, zr, 0)
        pltpu.sync_copy(r0, sacc.at[pl.ds(sid * SROWS, CB)])
        pltpu.sync_copy(r0, sacc.at[pl.ds(sid * SROWS + CB, CB)])
        pltpu.sync_copy(r0.at[pl.ds(0, SROWS - 2 * CB)],
                        sacc.at[pl.ds(sid * SROWS + 2 * CB,
                                      SROWS - 2 * CB)])

        hoff = jnp.full((16,), h * NPAD, jnp.int32)

        def mkidx(ci, b):
            cbase = ci * CB
            def g1(g, _):
                sv = plsc.load_gather(srcb, [cbase + g * 16 + ii])
                igs[b][pl.ds(g * 16, 16)] = (
                    jnp.minimum(sv, NPAD - 1) + hoff)
                lv = jnp.minimum(dstb[pl.ds(cbase + g * 16, 16)], nbv)
                iss[b][pl.ds(g * 16, 16)] = lv + soff
                return 0
            lax.fori_loop(0, CB // 16, g1, 0)

        for p in range(2):
            mkidx(p, p)
            pltpu.async_copy(h_t.at[igs[p]], rows[p], sgs[p])

        def p3(ci3, _):
            for b3 in range(3):
                ci = ci3 * 3 + b3
                b = b3

                @pl.when(ci < nc2)
                def _():
                    nb_ = (b3 + 2) % 3

                    @pl.when((ci >= 1) & (ci + 2 < nc2))
                    def _():
                        pltpu.make_async_copy(
                            rows[nb_], sacc.at[iss[nb_]],
                            sss[nb_]).wait()

                    @pl.when(ci + 2 < nc2)
                    def _():
                        mkidx(ci + 2, nb_)
                        pltpu.async_copy(h_t.at[igs[nb_]], rows[nb_],
                                         sgs[nb_])
                    pltpu.make_async_copy(h_t.at[igs[b]], rows[b],
                                          sgs[b]).wait()

                    def edge(j2, _):
                        for u in range(2):
                            j = j2 * 2 + u
                            jv = jnp.full((16,), ci * CB + j, jnp.int32)
                            av = plsc.load_gather(attnb, [jv])
                            for c in range(D // 16):
                                rv = rows[b][j, pl.ds(c * 16, 16)]
                                rows[b][j, pl.ds(c * 16, 16)] = rv * av
                        return 0
                    lax.fori_loop(0, CB // 2, edge, 0)
                    pltpu.async_copy(rows[b], sacc.at[iss[b]], sss[b],
                                     add=True)
            return 0
        lax.fori_loop(0, (nc2 + 2) // 3, p3, 0)

        for b in range(3):
            pltpu.make_async_copy(rows[b], sacc.at[iss[b]],
                                  sss[b]).wait()

        for seg, ln in ((0, CB), (1, CB), (2, NB - 2 * CB)):
            pltpu.sync_copy(
                sacc.at[pl.ds(sid * SROWS + seg * CB, ln)],
                r1.at[pl.ds(0, ln)])
            pltpu.sync_copy(
                r1.at[pl.ds(0, ln)],
                agg.at[pl.ds(n0 + seg * CB, ln), pl.ds(h * DR, DR)])


def _edge_layer(H, D, h_t, asrc_t, adst_t, srcs, dsts, bnds):
    DR = h_t.shape[1]
    kern = pl.kernel(
        functools.partial(_edge_body, H, D, DR),
        out_type=jax.ShapeDtypeStruct((NPAD, H * DR), jnp.float32),
        mesh=_mesh(),
        compiler_params=pltpu.CompilerParams(needs_layout_passes=False),
        scratch_types=[
            pltpu.VMEM((48,), jnp.int32),
            pltpu.VMEM((KMAXS + 128,), jnp.int32),
            pltpu.VMEM((KMAXS + 128,), jnp.int32),
            pltpu.VMEM((NPAD,), jnp.float32),
            pltpu.VMEM((NB + 16,), jnp.float32),
            pltpu.VMEM((NB + 16,), jnp.float32),
            pltpu.VMEM((KMAXS + 128,), jnp.float32),
            pltpu.VMEM((CB,), jnp.int32),
            pltpu.VMEM((CB,), jnp.int32),
            pltpu.VMEM((CB,), jnp.int32),
            pltpu.VMEM((CB,), jnp.int32),
            pltpu.VMEM((CB,), jnp.int32),
            pltpu.VMEM((CB,), jnp.int32),
            pltpu.VMEM((CB, DR), jnp.float32),
            pltpu.VMEM((CB, DR), jnp.float32),
            pltpu.VMEM((CB, DR), jnp.float32),
            pltpu.VMEM_SHARED((16 * SROWS, DR), jnp.float32),
            pltpu.SemaphoreType.DMA,
            pltpu.SemaphoreType.DMA,
            pltpu.SemaphoreType.DMA,
            pltpu.SemaphoreType.DMA,
            pltpu.SemaphoreType.DMA,
            pltpu.SemaphoreType.DMA,
        ],
    )
    return kern(h_t, asrc_t, adst_t, srcs, dsts, bnds)


def kernel(x, edge_index, W1, a_src1, a_dst1, b1, W2, a_src2, a_dst2, b2,
           W3, a_src3, a_dst3, b3, Wr, br):
    src = edge_index[0].astype(jnp.int32)
    dst = edge_index[1].astype(jnp.int32)
    order = jnp.argsort(dst)
    srcs = jnp.zeros((EPAD,), jnp.int32).at[:E].set(src[order])
    dsts = jnp.full((EPAD,), NPAD + 7, jnp.int32).at[:E].set(dst[order])
    starts = jnp.arange(NW + 1, dtype=jnp.int32) * NB
    bnds = jnp.searchsorted(dsts[:E], starts).astype(jnp.int32)
    bnds = jnp.concatenate([bnds, jnp.full((48 - NW - 1,), E, jnp.int32)])

    xp = jnp.zeros((NPAD, D_IN), jnp.float32).at[:N].set(x)

    h1, as1, ad1 = _dense_layer(xp, jnp.zeros((D_IN,), jnp.float32), W1,
                                a_src1, a_dst1, HEADS, HID, False)
    agg1 = _edge_layer(HEADS, HID, h1.reshape(HEADS * NPAD, HID),
                       as1.reshape(HEADS * NPAD), ad1.reshape(HEADS * NPAD),
                       srcs, dsts, bnds)

    h2, as2, ad2 = _dense_layer(agg1, b1, W2, a_src2, a_dst2, 1, HID, True)
    agg2 = _edge_layer(1, HID, h2.reshape(NPAD, HID),
                       as2.reshape(NPAD), ad2.reshape(NPAD),
                       srcs, dsts, bnds)

    h3, as3, ad3 = _dense_layer(agg2, b2, W3, a_src3, a_dst3, 1, HID // 2,
                                True)
    h3p = jnp.pad(h3.reshape(NPAD, HID // 2), ((0, 0), (0, HID // 2)))
    agg3 = _edge_layer(1, HID // 2, h3p,
                       as3.reshape(NPAD), ad3.reshape(NPAD),
                       srcs, dsts, bnds)

    agg3 = agg3[:, :HID // 2]

    y = _final_layer(agg3, b3, Wr, br)
    return y[:N]


# speedup vs baseline: 22.7060x; 1.0054x over previous
import functools

import jax
import jax.numpy as jnp
from jax import lax
from jax.experimental import pallas as pl
from jax.experimental.pallas import tpu as pltpu
from jax.experimental.pallas import tpu_sc as plsc

N = 10000
E = 320000
D_IN = 128
HID = 128
HEADS = 8

NW = 32
NB = 320
NPAD = NW * NB
BN = 512
NBLK = NPAD // BN
KMAXS = 11136
CB2 = 256
EPAD = E + KMAXS + 160

def _mesh():
    return plsc.VectorSubcoreMesh(core_axis_name="c", subcore_axis_name="s")


def _dense_body(apply_elu, x_ref, b_ref, W_ref, a_s_ref, a_d_ref,
                h_ref, as_ref, ad_ref):
    xb = x_ref[...]
    if apply_elu:
        xb = xb + b_ref[0]
        xb = jnp.where(xb > 0, xb, jnp.exp(jnp.minimum(xb, 0.0)) - 1.0)
    h = jnp.dot(xb, W_ref[...], preferred_element_type=jnp.float32)
    h_ref[0] = h
    as_ref[0, 0] = jnp.sum(h * a_s_ref[0], axis=1)
    ad_ref[0, 0] = jnp.sum(h * a_d_ref[0], axis=1)


def _dense_layer(x, b_prev, W, a_s, a_d, heads, dout, apply_elu):
    din = x.shape[1]
    grid = (NBLK, heads)
    out_shapes = [
        jax.ShapeDtypeStruct((heads, NPAD, dout), jnp.float32),
        jax.ShapeDtypeStruct((heads, 1, NPAD), jnp.float32),
        jax.ShapeDtypeStruct((heads, 1, NPAD), jnp.float32),
    ]
    fn = pl.pallas_call(
        functools.partial(_dense_body, apply_elu),
        grid=grid,
        in_specs=[
            pl.BlockSpec((BN, din), lambda i, h: (i, 0)),
            pl.BlockSpec((1, din), lambda i, h: (0, 0)),
            pl.BlockSpec((din, dout), lambda i, h: (0, h)),
            pl.BlockSpec((1, 1, dout), lambda i, h: (h, 0, 0)),
            pl.BlockSpec((1, 1, dout), lambda i, h: (h, 0, 0)),
        ],
        out_specs=[
            pl.BlockSpec((1, BN, dout), lambda i, h: (h, i, 0)),
            pl.BlockSpec((1, 1, BN), lambda i, h: (h, 0, i)),
            pl.BlockSpec((1, 1, BN), lambda i, h: (h, 0, i)),
        ],
        out_shape=out_shapes,
    )
    return fn(x, b_prev.reshape(1, din), W, a_s.reshape(heads, 1, dout),
              a_d.reshape(heads, 1, dout))


def _final_body(x_ref, b_ref, w_ref, br_ref, y_ref):
    xb = x_ref[...] + b_ref[0]
    xb = jnp.where(xb > 0, xb, jnp.exp(jnp.minimum(xb, 0.0)) - 1.0)
    y = jnp.sum(xb * w_ref[0], axis=1) + br_ref[0, 0]
    y_ref[0, 0] = jax.nn.sigmoid(y)


def _final_layer(x, b_prev, Wr, br):
    din = x.shape[1]
    fn = pl.pallas_call(
        _final_body,
        grid=(NBLK,),
        in_specs=[
            pl.BlockSpec((BN, din), lambda i: (i, 0)),
            pl.BlockSpec((1, din), lambda i: (0, 0)),
            pl.BlockSpec((1, din), lambda i: (0, 0)),
            pl.BlockSpec((1, 1), lambda i: (0, 0)),
        ],
        out_specs=pl.BlockSpec((1, 1, BN), lambda i: (i, 0, 0)),
        out_shape=jax.ShapeDtypeStruct((NBLK, 1, BN), jnp.float32),
    )
    y = fn(x, b_prev.reshape(1, din), Wr.reshape(1, din),
           br.reshape(1, 1))
    return y.reshape(NPAD)


CB = 112
SROWS = NB + 8


def _edge_body(H, D, DR, h_t, asrc_t, adst_t, srcs, dsts, bnds, agg,
               bv, srcb, dstb, asrc_v, adst_v, den_v, attnb,
               ig0, ig1, ig2, is0, is1, is2, r0, r1, r2, sacc,
               sg0, sg1, sg2, ss0, ss1, ss2):
    igs = (ig0, ig1, ig2)
    iss = (is0, is1, is2)
    rows = (r0, r1, r2)
    sgs = (sg0, sg1, sg2)
    sss = (ss0, ss1, ss2)
    wid = lax.axis_index("s") * 2 + lax.axis_index("c")
    sid = lax.axis_index("s")
    n0 = wid * NB
    pltpu.sync_copy(bnds, bv)
    ii = lax.iota(jnp.int32, 16)

    def _bnd(w):
        chunk = bv[pl.ds((w // 16) * 16, 16)]
        return jnp.sum(jnp.where(ii == w % 16, chunk, 0))

    e0 = _bnd(wid)
    e1 = _bnd(wid + 1)
    ea = (e0 // 8) * 8
    k = e1 - ea
    ng = (k + 15) // 16
    nc2 = (k + CB - 1) // CB

    pltpu.sync_copy(srcs.at[pl.ds(ea, KMAXS + 128)], srcb)
    pltpu.sync_copy(dsts.at[pl.ds(ea, KMAXS + 128)], dstb)

    e0v = jnp.full((16,), e0, jnp.int32)
    e1v = jnp.full((16,), e1, jnp.int32)
    n0v = jnp.full((16,), n0, jnp.int32)
    nbm1 = jnp.full((16,), NB - 1, jnp.int32)
    nbv = jnp.full((16,), NB, jnp.int32)
    soff = jnp.full((16,), sid * SROWS, jnp.int32)

    def enc(g, _):
        base = g * 16
        dv = dstb[pl.ds(base, 16)]
        gev = jnp.full((16,), ea + base, jnp.int32) + ii
        active = (gev >= e0v) & (gev < e1v)
        ldst = jnp.minimum(jnp.maximum(dv - n0v, 0), nbm1)
        dstb[pl.ds(base, 16)] = jnp.where(active, ldst, nbv)
        return 0
    lax.fori_loop(0, (KMAXS + 16) // 16, enc, 0)

    for h in range(H):
        pltpu.sync_copy(asrc_t.at[pl.ds(h * NPAD, NPAD)], asrc_v)
        pltpu.sync_copy(adst_t.at[pl.ds(h * NPAD + n0, NB)],
                        adst_v.at[pl.ds(0, NB)])

        def zden(i, _):
            den_v[pl.ds(i * 16, 16)] = jnp.zeros((16,), jnp.float32)
            return 0
        lax.fori_loop(0, (NB + 16) // 16, zden, 0)

        def logits(g):
            base = g * 16
            ldst = dstb[pl.ds(base, 16)]
            active = ldst < nbv
            sv = plsc.load_gather(srcb, [base + ii])
            asv = plsc.load_gather(asrc_v, [jnp.minimum(sv, NPAD - 1)])
            adv = plsc.load_gather(adst_v, [ldst])
            a = asv + adv
            a = jnp.where(a > 0, a, 0.2 * a)
            ex = jnp.where(active, jnp.exp(a), 0.0)
            return ldst, active, ex

        def p1(g, _):
            ldst, active, ex = logits(g)
            attnb[pl.ds(g * 16, 16)] = ex
            cs = plsc.cumsum(ex)
            nldst = plsc.load_gather(dstb, [g * 16 + ii + 1])
            run_end = (ldst != nldst) | (ii == 15)
            plsc.addupdate_scatter(den_v, [ldst], cs,
                                   mask=run_end & active)
            plsc.addupdate_scatter(den_v, [nldst], -cs,
                                   mask=run_end & active & (nldst < nbv)
                                   & (ii < 15))
            return 0
        lax.fori_loop(0, ng, p1, 0)

        def p2(g, _):
            base = g * 16
            ldst = dstb[pl.ds(base, 16)]
            ex = attnb[pl.ds(base, 16)]
            dd = plsc.load_gather(den_v, [ldst])
            attnb[pl.ds(base, 16)] = ex / jnp.where(ldst < nbv, dd, 1.0)
            return 0
        lax.fori_loop(0, ng, p2, 0)

        def zr(i, _):
            for c in range(DR // 16):
                r0[i, pl.ds(c * 16, 16)] = jnp.zeros((16,), jnp.float32)
            return 0
        lax.fori_loop(0, CB, zr, 0)
        pltpu.sync_copy(r0, sacc.at[pl.ds(sid * SROWS, CB)])
        pltpu.sync_copy(r0, sacc.at[pl.ds(sid * SROWS + CB, CB)])
        pltpu.sync_copy(r0.at[pl.ds(0, SROWS - 2 * CB)],
                        sacc.at[pl.ds(sid * SROWS + 2 * CB,
                                      SROWS - 2 * CB)])

        hoff = jnp.full((16,), h * NPAD, jnp.int32)

        def mkidx(ci, b):
            cbase = ci * CB
            def g1(g, _):
                sv = plsc.load_gather(srcb, [cbase + g * 16 + ii])
                igs[b][pl.ds(g * 16, 16)] = (
                    jnp.minimum(sv, NPAD - 1) + hoff)
                lv = jnp.minimum(dstb[pl.ds(cbase + g * 16, 16)], nbv)
                iss[b][pl.ds(g * 16, 16)] = lv + soff
                return 0
            lax.fori_loop(0, CB // 16, g1, 0)

        for p in range(2):
            mkidx(p, p)
            pltpu.async_copy(h_t.at[igs[p]], rows[p], sgs[p])

        def p3(ci3, _):
            for b3 in range(3):
                ci = ci3 * 3 + b3
                b = b3

                @pl.when(ci < nc2)
                def _():
                    nb_ = (b3 + 2) % 3

                    @pl.when((ci >= 1) & (ci + 2 < nc2))
                    def _():
                        pltpu.make_async_copy(
                            rows[nb_], sacc.at[iss[nb_]],
                            sss[nb_]).wait()

                    @pl.when(ci + 2 < nc2)
                    def _():
                        mkidx(ci + 2, nb_)
                        pltpu.async_copy(h_t.at[igs[nb_]], rows[nb_],
                                         sgs[nb_])
                    pltpu.make_async_copy(h_t.at[igs[b]], rows[b],
                                          sgs[b]).wait()

                    def edge(j2, _):
                        for u in range(4):
                            j = j2 * 4 + u
                            jv = jnp.full((16,), ci * CB + j, jnp.int32)
                            av = plsc.load_gather(attnb, [jv])
                            for c in range(D // 16):
                                rv = rows[b][j, pl.ds(c * 16, 16)]
                                rows[b][j, pl.ds(c * 16, 16)] = rv * av
                        return 0
                    lax.fori_loop(0, CB // 4, edge, 0)
                    pltpu.async_copy(rows[b], sacc.at[iss[b]], sss[b],
                                     add=True)
            return 0
        lax.fori_loop(0, (nc2 + 2) // 3, p3, 0)

        for b in range(3):
            pltpu.make_async_copy(rows[b], sacc.at[iss[b]],
                                  sss[b]).wait()

        for seg, ln in ((0, CB), (1, CB), (2, NB - 2 * CB)):
            pltpu.sync_copy(
                sacc.at[pl.ds(sid * SROWS + seg * CB, ln)],
                r1.at[pl.ds(0, ln)])
            pltpu.sync_copy(
                r1.at[pl.ds(0, ln)],
                agg.at[pl.ds(n0 + seg * CB, ln), pl.ds(h * DR, DR)])


def _edge_layer(H, D, h_t, asrc_t, adst_t, srcs, dsts, bnds):
    DR = h_t.shape[1]
    kern = pl.kernel(
        functools.partial(_edge_body, H, D, DR),
        out_type=jax.ShapeDtypeStruct((NPAD, H * DR), jnp.float32),
        mesh=_mesh(),
        compiler_params=pltpu.CompilerParams(needs_layout_passes=False),
        scratch_types=[
            pltpu.VMEM((48,), jnp.int32),
            pltpu.VMEM((KMAXS + 128,), jnp.int32),
            pltpu.VMEM((KMAXS + 128,), jnp.int32),
            pltpu.VMEM((NPAD,), jnp.float32),
            pltpu.VMEM((NB + 16,), jnp.float32),
            pltpu.VMEM((NB + 16,), jnp.float32),
            pltpu.VMEM((KMAXS + 128,), jnp.float32),
            pltpu.VMEM((CB,), jnp.int32),
            pltpu.VMEM((CB,), jnp.int32),
            pltpu.VMEM((CB,), jnp.int32),
            pltpu.VMEM((CB,), jnp.int32),
            pltpu.VMEM((CB,), jnp.int32),
            pltpu.VMEM((CB,), jnp.int32),
            pltpu.VMEM((CB, DR), jnp.float32),
            pltpu.VMEM((CB, DR), jnp.float32),
            pltpu.VMEM((CB, DR), jnp.float32),
            pltpu.VMEM_SHARED((16 * SROWS, DR), jnp.float32),
            pltpu.SemaphoreType.DMA,
            pltpu.SemaphoreType.DMA,
            pltpu.SemaphoreType.DMA,
            pltpu.SemaphoreType.DMA,
            pltpu.SemaphoreType.DMA,
            pltpu.SemaphoreType.DMA,
        ],
    )
    return kern(h_t, asrc_t, adst_t, srcs, dsts, bnds)


def kernel(x, edge_index, W1, a_src1, a_dst1, b1, W2, a_src2, a_dst2, b2,
           W3, a_src3, a_dst3, b3, Wr, br):
    src = edge_index[0].astype(jnp.int32)
    dst = edge_index[1].astype(jnp.int32)
    order = jnp.argsort(dst)
    srcs = jnp.zeros((EPAD,), jnp.int32).at[:E].set(src[order])
    dsts = jnp.full((EPAD,), NPAD + 7, jnp.int32).at[:E].set(dst[order])
    starts = jnp.arange(NW + 1, dtype=jnp.int32) * NB
    bnds = jnp.searchsorted(dsts[:E], starts).astype(jnp.int32)
    bnds = jnp.concatenate([bnds, jnp.full((48 - NW - 1,), E, jnp.int32)])

    xp = jnp.zeros((NPAD, D_IN), jnp.float32).at[:N].set(x)

    h1, as1, ad1 = _dense_layer(xp, jnp.zeros((D_IN,), jnp.float32), W1,
                                a_src1, a_dst1, HEADS, HID, False)
    agg1 = _edge_layer(HEADS, HID, h1.reshape(HEADS * NPAD, HID),
                       as1.reshape(HEADS * NPAD), ad1.reshape(HEADS * NPAD),
                       srcs, dsts, bnds)

    h2, as2, ad2 = _dense_layer(agg1, b1, W2, a_src2, a_dst2, 1, HID, True)
    agg2 = _edge_layer(1, HID, h2.reshape(NPAD, HID),
                       as2.reshape(NPAD), ad2.reshape(NPAD),
                       srcs, dsts, bnds)

    h3, as3, ad3 = _dense_layer(agg2, b2, W3, a_src3, a_dst3, 1, HID // 2,
                                True)
    h3p = jnp.pad(h3.reshape(NPAD, HID // 2), ((0, 0), (0, HID // 2)))
    agg3 = _edge_layer(1, HID // 2, h3p,
                       as3.reshape(NPAD), ad3.reshape(NPAD),
                       srcs, dsts, bnds)

    agg3 = agg3[:, :HID // 2]

    y = _final_layer(agg3, b3, Wr, br)
    return y[:N]


# speedup vs baseline: 22.7158x; 1.0004x over previous
import functools

import jax
import jax.numpy as jnp
from jax import lax
from jax.experimental import pallas as pl
from jax.experimental.pallas import tpu as pltpu
from jax.experimental.pallas import tpu_sc as plsc

N = 10000
E = 320000
D_IN = 128
HID = 128
HEADS = 8

NW = 32
NB = 320
NPAD = NW * NB
BN = 512
NBLK = NPAD // BN
KMAXS = 11136
EPAD = E + KMAXS + 160

def _mesh():
    return plsc.VectorSubcoreMesh(core_axis_name="c", subcore_axis_name="s")


def _dense_body(apply_elu, x_ref, b_ref, W_ref, a_s_ref, a_d_ref,
                h_ref, as_ref, ad_ref):
    xb = x_ref[...]
    if apply_elu:
        xb = xb + b_ref[0]
        xb = jnp.where(xb > 0, xb, jnp.exp(jnp.minimum(xb, 0.0)) - 1.0)
    h = jnp.dot(xb, W_ref[...], preferred_element_type=jnp.float32)
    h_ref[0] = h
    as_ref[0, 0] = jnp.sum(h * a_s_ref[0], axis=1)
    ad_ref[0, 0] = jnp.sum(h * a_d_ref[0], axis=1)


def _dense_layer(x, b_prev, W, a_s, a_d, heads, dout, apply_elu):
    din = x.shape[1]
    grid = (NBLK, heads)
    out_shapes = [
        jax.ShapeDtypeStruct((heads, NPAD, dout), jnp.float32),
        jax.ShapeDtypeStruct((heads, 1, NPAD), jnp.float32),
        jax.ShapeDtypeStruct((heads, 1, NPAD), jnp.float32),
    ]
    fn = pl.pallas_call(
        functools.partial(_dense_body, apply_elu),
        grid=grid,
        in_specs=[
            pl.BlockSpec((BN, din), lambda i, h: (i, 0)),
            pl.BlockSpec((1, din), lambda i, h: (0, 0)),
            pl.BlockSpec((din, dout), lambda i, h: (0, h)),
            pl.BlockSpec((1, 1, dout), lambda i, h: (h, 0, 0)),
            pl.BlockSpec((1, 1, dout), lambda i, h: (h, 0, 0)),
        ],
        out_specs=[
            pl.BlockSpec((1, BN, dout), lambda i, h: (h, i, 0)),
            pl.BlockSpec((1, 1, BN), lambda i, h: (h, 0, i)),
            pl.BlockSpec((1, 1, BN), lambda i, h: (h, 0, i)),
        ],
        out_shape=out_shapes,
    )
    return fn(x, b_prev.reshape(1, din), W, a_s.reshape(heads, 1, dout),
              a_d.reshape(heads, 1, dout))


def _final_body(x_ref, b_ref, w_ref, br_ref, y_ref):
    xb = x_ref[...] + b_ref[0]
    xb = jnp.where(xb > 0, xb, jnp.exp(jnp.minimum(xb, 0.0)) - 1.0)
    y = jnp.sum(xb * w_ref[0], axis=1) + br_ref[0, 0]
    y_ref[0, 0] = jax.nn.sigmoid(y)


def _final_layer(x, b_prev, Wr, br):
    din = x.shape[1]
    fn = pl.pallas_call(
        _final_body,
        grid=(NBLK,),
        in_specs=[
            pl.BlockSpec((BN, din), lambda i: (i, 0)),
            pl.BlockSpec((1, din), lambda i: (0, 0)),
            pl.BlockSpec((1, din), lambda i: (0, 0)),
            pl.BlockSpec((1, 1), lambda i: (0, 0)),
        ],
        out_specs=pl.BlockSpec((1, 1, BN), lambda i: (i, 0, 0)),
        out_shape=jax.ShapeDtypeStruct((NBLK, 1, BN), jnp.float32),
    )
    y = fn(x, b_prev.reshape(1, din), Wr.reshape(1, din),
           br.reshape(1, 1))
    return y.reshape(NPAD)


CB = 112
SROWS = NB + 8


def _edge_body(H, D, DR, h_t, asrc_t, adst_t, srcs, dsts, bnds, agg,
               bv, srcb, dstb, asrc_v, adst_v, den_v, attnb,
               ig0, ig1, ig2, is0, is1, is2, r0, r1, r2, sacc,
               sg0, sg1, sg2, ss0, ss1, ss2):
    igs = (ig0, ig1, ig2)
    iss = (is0, is1, is2)
    rows = (r0, r1, r2)
    sgs = (sg0, sg1, sg2)
    sss = (ss0, ss1, ss2)
    wid = lax.axis_index("s") * 2 + lax.axis_index("c")
    sid = lax.axis_index("s")
    n0 = wid * NB
    pltpu.sync_copy(bnds, bv)
    ii = lax.iota(jnp.int32, 16)

    def _bnd(w):
        chunk = bv[pl.ds((w // 16) * 16, 16)]
        return jnp.sum(jnp.where(ii == w % 16, chunk, 0))

    e0 = _bnd(wid)
    e1 = _bnd(wid + 1)
    ea = (e0 // 8) * 8
    k = e1 - ea
    ng = (k + 15) // 16
    nc2 = (k + CB - 1) // CB

    pltpu.sync_copy(srcs.at[pl.ds(ea, KMAXS + 128)], srcb)
    pltpu.sync_copy(dsts.at[pl.ds(ea, KMAXS + 128)], dstb)

    e0v = jnp.full((16,), e0, jnp.int32)
    e1v = jnp.full((16,), e1, jnp.int32)
    n0v = jnp.full((16,), n0, jnp.int32)
    nbm1 = jnp.full((16,), NB - 1, jnp.int32)
    nbv = jnp.full((16,), NB, jnp.int32)
    soff = jnp.full((16,), sid * SROWS, jnp.int32)

    def enc(g, _):
        base = g * 16
        dv = dstb[pl.ds(base, 16)]
        gev = jnp.full((16,), ea + base, jnp.int32) + ii
        active = (gev >= e0v) & (gev < e1v)
        ldst = jnp.minimum(jnp.maximum(dv - n0v, 0), nbm1)
        dstb[pl.ds(base, 16)] = jnp.where(active, ldst, nbv)
        return 0
    lax.fori_loop(0, (KMAXS + 16) // 16, enc, 0)

    for h in range(H):
        pltpu.sync_copy(asrc_t.at[pl.ds(h * NPAD, NPAD)], asrc_v)
        pltpu.sync_copy(adst_t.at[pl.ds(h * NPAD + n0, NB)],
                        adst_v.at[pl.ds(0, NB)])

        def zden(i, _):
            den_v[pl.ds(i * 16, 16)] = jnp.zeros((16,), jnp.float32)
            return 0
        lax.fori_loop(0, (NB + 16) // 16, zden, 0)

        def logits(g):
            base = g * 16
            ldst = dstb[pl.ds(base, 16)]
            active = ldst < nbv
            sv = plsc.load_gather(srcb, [base + ii])
            asv = plsc.load_gather(asrc_v, [jnp.minimum(sv, NPAD - 1)])
            adv = plsc.load_gather(adst_v, [ldst])
            a = asv + adv
            a = jnp.where(a > 0, a, 0.2 * a)
            ex = jnp.where(active, jnp.exp(a), 0.0)
            return ldst, active, ex

        def p1(g, _):
            ldst, active, ex = logits(g)
            attnb[pl.ds(g * 16, 16)] = ex
            cs = plsc.cumsum(ex)
            nldst = plsc.load_gather(dstb, [g * 16 + ii + 1])
            run_end = (ldst != nldst) | (ii == 15)
            plsc.addupdate_scatter(den_v, [ldst], cs,
                                   mask=run_end & active)
            plsc.addupdate_scatter(den_v, [nldst], -cs,
                                   mask=run_end & active & (nldst < nbv)
                                   & (ii < 15))
            return 0
        lax.fori_loop(0, ng, p1, 0)

        def p2(g, _):
            base = g * 16
            ldst = dstb[pl.ds(base, 16)]
            ex = attnb[pl.ds(base, 16)]
            dd = plsc.load_gather(den_v, [ldst])
            attnb[pl.ds(base, 16)] = ex / jnp.where(ldst < nbv, dd, 1.0)
            return 0
        lax.fori_loop(0, ng, p2, 0)

        def zr(i, _):
            for c in range(DR // 16):
                r0[i, pl.ds(c * 16, 16)] = jnp.zeros((16,), jnp.float32)
            return 0
        lax.fori_loop(0, CB, zr, 0)
        pltpu.sync_copy(r0, sacc.at[pl.ds(sid * SROWS, CB)])
        pltpu.sync_copy(r0, sacc.at[pl.ds(sid * SROWS + CB, CB)])
        pltpu.sync_copy(r0.at[pl.ds(0, SROWS - 2 * CB)],
                        sacc.at[pl.ds(sid * SROWS + 2 * CB,
                                      SROWS - 2 * CB)])

        hoff = jnp.full((16,), h * NPAD, jnp.int32)

        def mkidx(ci, b):
            cbase = ci * CB
            def g1(g, _):
                sv = plsc.load_gather(srcb, [cbase + g * 16 + ii])
                igs[b][pl.ds(g * 16, 16)] = (
                    jnp.minimum(sv, NPAD - 1) + hoff)
                lv = jnp.minimum(dstb[pl.ds(cbase + g * 16, 16)], nbv)
                iss[b][pl.ds(g * 16, 16)] = lv + soff
                return 0
            lax.fori_loop(0, CB // 16, g1, 0)

        for p in range(2):
            mkidx(p, p)
            pltpu.async_copy(h_t.at[igs[p]], rows[p], sgs[p])

        def p3(ci3, _):
            for b3 in range(3):
                ci = ci3 * 3 + b3
                b = b3

                @pl.when(ci < nc2)
                def _():
                    nb_ = (b3 + 2) % 3

                    @pl.when((ci >= 1) & (ci + 2 < nc2))
                    def _():
                        pltpu.make_async_copy(
                            rows[nb_], sacc.at[iss[nb_]],
                            sss[nb_]).wait()

                    @pl.when(ci + 2 < nc2)
                    def _():
                        mkidx(ci + 2, nb_)
                        pltpu.async_copy(h_t.at[igs[nb_]], rows[nb_],
                                         sgs[nb_])
                    pltpu.make_async_copy(h_t.at[igs[b]], rows[b],
                                          sgs[b]).wait()

                    def edge(j2, _):
                        for u in range(4):
                            j = j2 * 4 + u
                            jv = jnp.full((16,), ci * CB + j, jnp.int32)
                            av = plsc.load_gather(attnb, [jv])
                            for c in range(D // 16):
                                rv = rows[b][j, pl.ds(c * 16, 16)]
                                rows[b][j, pl.ds(c * 16, 16)] = rv * av
                        return 0
                    lax.fori_loop(0, CB // 4, edge, 0)
                    pltpu.async_copy(rows[b], sacc.at[iss[b]], sss[b],
                                     add=True)
            return 0
        lax.fori_loop(0, (nc2 + 2) // 3, p3, 0)

        for b in range(3):
            pltpu.make_async_copy(rows[b], sacc.at[iss[b]],
                                  sss[b]).wait()

        for seg, ln in ((0, CB), (1, CB), (2, NB - 2 * CB)):
            pltpu.sync_copy(
                sacc.at[pl.ds(sid * SROWS + seg * CB, ln)],
                r1.at[pl.ds(0, ln)])
            pltpu.sync_copy(
                r1.at[pl.ds(0, ln)],
                agg.at[pl.ds(n0 + seg * CB, ln), pl.ds(h * DR, DR)])


def _edge_layer(H, D, h_t, asrc_t, adst_t, srcs, dsts, bnds):
    DR = h_t.shape[1]
    kern = pl.kernel(
        functools.partial(_edge_body, H, D, DR),
        out_type=jax.ShapeDtypeStruct((NPAD, H * DR), jnp.float32),
        mesh=_mesh(),
        compiler_params=pltpu.CompilerParams(needs_layout_passes=False),
        scratch_types=[
            pltpu.VMEM((48,), jnp.int32),
            pltpu.VMEM((KMAXS + 128,), jnp.int32),
            pltpu.VMEM((KMAXS + 128,), jnp.int32),
            pltpu.VMEM((NPAD,), jnp.float32),
            pltpu.VMEM((NB + 16,), jnp.float32),
            pltpu.VMEM((NB + 16,), jnp.float32),
            pltpu.VMEM((KMAXS + 128,), jnp.float32),
            pltpu.VMEM((CB,), jnp.int32),
            pltpu.VMEM((CB,), jnp.int32),
            pltpu.VMEM((CB,), jnp.int32),
            pltpu.VMEM((CB,), jnp.int32),
            pltpu.VMEM((CB,), jnp.int32),
            pltpu.VMEM((CB,), jnp.int32),
            pltpu.VMEM((CB, DR), jnp.float32),
            pltpu.VMEM((CB, DR), jnp.float32),
            pltpu.VMEM((CB, DR), jnp.float32),
            pltpu.VMEM_SHARED((16 * SROWS, DR), jnp.float32),
            pltpu.SemaphoreType.DMA,
            pltpu.SemaphoreType.DMA,
            pltpu.SemaphoreType.DMA,
            pltpu.SemaphoreType.DMA,
            pltpu.SemaphoreType.DMA,
            pltpu.SemaphoreType.DMA,
        ],
    )
    return kern(h_t, asrc_t, adst_t, srcs, dsts, bnds)


def kernel(x, edge_index, W1, a_src1, a_dst1, b1, W2, a_src2, a_dst2, b2,
           W3, a_src3, a_dst3, b3, Wr, br):
    src = edge_index[0].astype(jnp.int32)
    dst = edge_index[1].astype(jnp.int32)
    order = jnp.argsort(dst)
    srcs = jnp.zeros((EPAD,), jnp.int32).at[:E].set(src[order])
    dsts = jnp.full((EPAD,), NPAD + 7, jnp.int32).at[:E].set(dst[order])
    starts = jnp.arange(NW + 1, dtype=jnp.int32) * NB
    bnds = jnp.searchsorted(dsts[:E], starts).astype(jnp.int32)
    bnds = jnp.concatenate([bnds, jnp.full((48 - NW - 1,), E, jnp.int32)])

    xp = jnp.zeros((NPAD, D_IN), jnp.float32).at[:N].set(x)

    h1, as1, ad1 = _dense_layer(xp, jnp.zeros((D_IN,), jnp.float32), W1,
                                a_src1, a_dst1, HEADS, HID, False)
    agg1 = _edge_layer(HEADS, HID, h1.reshape(HEADS * NPAD, HID),
                       as1.reshape(HEADS * NPAD), ad1.reshape(HEADS * NPAD),
                       srcs, dsts, bnds)

    h2, as2, ad2 = _dense_layer(agg1, b1, W2, a_src2, a_dst2, 1, HID, True)
    agg2 = _edge_layer(1, HID, h2.reshape(NPAD, HID),
                       as2.reshape(NPAD), ad2.reshape(NPAD),
                       srcs, dsts, bnds)

    h3, as3, ad3 = _dense_layer(agg2, b2, W3, a_src3, a_dst3, 1, HID // 2,
                                True)
    h3p = jnp.pad(h3.reshape(NPAD, HID // 2), ((0, 0), (0, HID // 2)))
    agg3 = _edge_layer(1, HID // 2, h3p,
                       as3.reshape(NPAD), ad3.reshape(NPAD),
                       srcs, dsts, bnds)

    agg3 = agg3[:, :HID // 2]

    y = _final_layer(agg3, b3, Wr, br)
    return y[:N]
